# Initial kernel scaffold; baseline (speedup 1.0000x reference)
import jax
import jax.numpy as jnp
from jax.experimental import pallas as pl


def kernel(ts, pos_embeddings):
    raise NotImplementedError("write your pallas kernel here")


# speedup vs baseline: 4.6339x; 4.6339x over previous
import jax
import jax.numpy as jnp
from jax.experimental import pallas as pl

_B, _S, _E = 16384, 200, 64
_R = 16


def _body(ts_ref, emb_ref, out_ref):
    ts = ts_ref[...]
    a = ts[:, :, None]
    b = ts[:, None, :]
    k_iota = jax.lax.broadcasted_iota(jnp.int32, (_R, _S, _S), 2)
    j_iota = jax.lax.broadcasted_iota(jnp.int32, (_R, _S, _S), 1)
    cmp = (b < a) | ((b == a) & (k_iota < j_iota))
    rank = jnp.sum(cmp.astype(jnp.int32), axis=2)
    i_iota = jax.lax.broadcasted_iota(jnp.int32, (_R, _S, _S), 1)
    p = (i_iota == rank[:, None, :]).astype(jnp.float32)
    out = jnp.dot(p.reshape(_R * _S, _S), emb_ref[...],
                  preferred_element_type=jnp.float32)
    out_ref[...] = out.reshape(_R, _S, _E)


def kernel(ts, pos_embeddings):
    return pl.pallas_call(
        _body,
        grid=(_B // _R,),
        in_specs=[
            pl.BlockSpec((_R, _S), lambda i: (i, 0)),
            pl.BlockSpec((_S, _E), lambda i: (0, 0)),
        ],
        out_specs=pl.BlockSpec((_R, _S, _E), lambda i: (i, 0, 0)),
        out_shape=jax.ShapeDtypeStruct((_B, _S, _E), jnp.float32),
    )(ts, pos_embeddings)
